# baseline (speedup 1.0000x reference)
import jax
import jax.numpy as jnp
from jax import lax
from jax.experimental import pallas as pl
from jax.experimental.pallas import tpu as pltpu

E = 8
D = 1024
H = 2048
O = 1024
T = 2048
G = 64


def _dot_nt(a, b):
    return lax.dot_general(a, b, (((1,), (1,)), ((), ())),
                           preferred_element_type=jnp.float32)


def _moe_body(x_ref, W1_ref, b1_ref, W2_ref, b2_ref, W3_ref, b3_ref,
              Wg1_ref, bg1_ref, Wg2_ref, bg2_ref, out_ref,
              gates_ref, h1_ref):
    e = pl.program_id(1)
    g = pl.program_id(2)

    @pl.when(jnp.logical_and(e == 0, g == 0))
    def _compute_gates():
        xb = x_ref[...]
        gh = jnp.maximum(_dot_nt(xb, Wg1_ref[...]) + bg1_ref[...], 0.0)
        logits = _dot_nt(gh, Wg2_ref[...]) + bg2_ref[...]
        m = jnp.max(logits, axis=1, keepdims=True)
        p = jnp.exp(logits - m)
        gates_ref[...] = p / jnp.sum(p, axis=1, keepdims=True)

    @pl.when(g == 0)
    def _layer1():
        h1_ref[...] = jnp.maximum(
            _dot_nt(x_ref[...], W1_ref[0]) + b1_ref[0], 0.0)

    h2g = jnp.maximum(_dot_nt(h1_ref[...], W2_ref[0]) + b2_ref[0], 0.0)
    og = _dot_nt(h2g, W3_ref[0])

    lane = lax.broadcasted_iota(jnp.int32, (1, E), 1)
    ge = jnp.sum(gates_ref[...] * (lane == e).astype(jnp.float32),
                 axis=1, keepdims=True)
    contrib = og * ge

    @pl.when(g == 0)
    def _add_b3():
        contrib_b = (og + b3_ref[0]) * ge

        @pl.when(e == 0)
        def _init():
            out_ref[...] = contrib_b

        @pl.when(e != 0)
        def _accum():
            out_ref[...] += contrib_b

    @pl.when(g != 0)
    def _accum_rest():
        out_ref[...] += contrib


def _moe_pallas(x, W1, b1, W2, b2, W3, b3, Wg1, bg1, Wg2, bg2,
                block_t=512, block_h=256, interpret=False):
    t_tokens = x.shape[0]
    nt = t_tokens // block_t
    ng = H // block_h
    b1r = b1[:, None, :]
    b2r = b2[:, None, :]
    b3r = b3[:, None, :]
    bg1r = bg1[None, :]
    bg2r = bg2[None, :]
    grid = (nt, E, ng)
    return pl.pallas_call(
        _moe_body,
        grid=grid,
        in_specs=[
            pl.BlockSpec((block_t, D), lambda t, e, g: (t, 0)),
            pl.BlockSpec((1, H, D), lambda t, e, g: (e, 0, 0)),
            pl.BlockSpec((1, 1, H), lambda t, e, g: (e, 0, 0)),
            pl.BlockSpec((1, block_h, H), lambda t, e, g: (e, g, 0)),
            pl.BlockSpec((1, 1, block_h), lambda t, e, g: (e, 0, g)),
            pl.BlockSpec((1, O, block_h), lambda t, e, g: (e, 0, g)),
            pl.BlockSpec((1, 1, O), lambda t, e, g: (e, 0, 0)),
            pl.BlockSpec((G, D), lambda t, e, g: (0, 0)),
            pl.BlockSpec((1, G), lambda t, e, g: (0, 0)),
            pl.BlockSpec((E, G), lambda t, e, g: (0, 0)),
            pl.BlockSpec((1, E), lambda t, e, g: (0, 0)),
        ],
        out_specs=pl.BlockSpec((block_t, O), lambda t, e, g: (t, 0)),
        out_shape=jax.ShapeDtypeStruct((t_tokens, O), jnp.float32),
        scratch_shapes=[
            pltpu.VMEM((block_t, E), jnp.float32),
            pltpu.VMEM((block_t, H), jnp.float32),
        ],
        interpret=interpret,
    )(x, W1, b1r, W2, b2r, W3, b3r, Wg1, bg1r, Wg2, bg2r)


@jax.jit
def kernel(x, W1, b1, W2, b2, W3, b3, Wg1, bg1, Wg2, bg2):
    return _moe_pallas(x, W1, b1, W2, b2, W3, b3, Wg1, bg1, Wg2, bg2)


# speedup vs baseline: 1.2670x; 1.2670x over previous
import jax
import jax.numpy as jnp
from jax import lax
from jax.experimental import pallas as pl
from jax.experimental.pallas import tpu as pltpu

E = 8
D = 1024
H = 2048
O = 1024
T = 2048
G = 64


def _dot_nt(a, b):
    return lax.dot_general(a, b, (((1,), (1,)), ((), ())),
                           preferred_element_type=jnp.float32)


def _moe_body(x_ref, W1_ref, b1_ref, W2_ref, b2_ref, W3_ref, b3_ref,
              Wg1_ref, bg1_ref, Wg2_ref, bg2_ref, out_ref,
              gates_ref, h1_ref):
    e = pl.program_id(1)
    g = pl.program_id(2)

    @pl.when(jnp.logical_and(e == 0, g == 0))
    def _compute_gates():
        xb = x_ref[...]
        gh = jnp.maximum(_dot_nt(xb, Wg1_ref[...]) + bg1_ref[...], 0.0)
        logits = _dot_nt(gh, Wg2_ref[...]) + bg2_ref[...]
        m = jnp.max(logits, axis=1, keepdims=True)
        p = jnp.exp(logits - m)
        gates_ref[...] = p / jnp.sum(p, axis=1, keepdims=True)

    @pl.when(g == 0)
    def _layer1():
        xb = x_ref[...].astype(jnp.bfloat16)
        W1b = W1_ref[0].astype(jnp.bfloat16)
        h1_ref[...] = jnp.maximum(
            _dot_nt(xb, W1b) + b1_ref[0], 0.0).astype(jnp.bfloat16)

    W2b = W2_ref[0].astype(jnp.bfloat16)
    h2g = jnp.maximum(_dot_nt(h1_ref[...], W2b) + b2_ref[0],
                      0.0).astype(jnp.bfloat16)
    og = _dot_nt(h2g, W3_ref[0].astype(jnp.bfloat16))

    lane = lax.broadcasted_iota(jnp.int32, (1, E), 1)
    ge = jnp.sum(gates_ref[...] * (lane == e).astype(jnp.float32),
                 axis=1, keepdims=True)
    contrib = og * ge

    @pl.when(g == 0)
    def _add_b3():
        contrib_b = (og + b3_ref[0]) * ge

        @pl.when(e == 0)
        def _init():
            out_ref[...] = contrib_b

        @pl.when(e != 0)
        def _accum():
            out_ref[...] += contrib_b

    @pl.when(g != 0)
    def _accum_rest():
        out_ref[...] += contrib


def _moe_pallas(x, W1, b1, W2, b2, W3, b3, Wg1, bg1, Wg2, bg2,
                block_t=1024, block_h=256, interpret=False):
    t_tokens = x.shape[0]
    nt = t_tokens // block_t
    ng = H // block_h
    b1r = b1[:, None, :]
    b2r = b2[:, None, :]
    b3r = b3[:, None, :]
    bg1r = bg1[None, :]
    bg2r = bg2[None, :]
    grid = (nt, E, ng)
    return pl.pallas_call(
        _moe_body,
        grid=grid,
        in_specs=[
            pl.BlockSpec((block_t, D), lambda t, e, g: (t, 0)),
            pl.BlockSpec((1, H, D), lambda t, e, g: (e, 0, 0)),
            pl.BlockSpec((1, 1, H), lambda t, e, g: (e, 0, 0)),
            pl.BlockSpec((1, block_h, H), lambda t, e, g: (e, g, 0)),
            pl.BlockSpec((1, 1, block_h), lambda t, e, g: (e, 0, g)),
            pl.BlockSpec((1, O, block_h), lambda t, e, g: (e, 0, g)),
            pl.BlockSpec((1, 1, O), lambda t, e, g: (e, 0, 0)),
            pl.BlockSpec((G, D), lambda t, e, g: (0, 0)),
            pl.BlockSpec((1, G), lambda t, e, g: (0, 0)),
            pl.BlockSpec((E, G), lambda t, e, g: (0, 0)),
            pl.BlockSpec((1, E), lambda t, e, g: (0, 0)),
        ],
        out_specs=pl.BlockSpec((block_t, O), lambda t, e, g: (t, 0)),
        out_shape=jax.ShapeDtypeStruct((t_tokens, O), jnp.float32),
        scratch_shapes=[
            pltpu.VMEM((block_t, E), jnp.float32),
            pltpu.VMEM((block_t, H), jnp.bfloat16),
        ],
        interpret=interpret,
    )(x, W1, b1r, W2, b2r, W3, b3r, Wg1, bg1r, Wg2, bg2r)


@jax.jit
def kernel(x, W1, b1, W2, b2, W3, b3, Wg1, bg1, Wg2, bg2):
    return _moe_pallas(x, W1, b1, W2, b2, W3, b3, Wg1, bg1, Wg2, bg2)


# speedup vs baseline: 1.6327x; 1.2886x over previous
import jax
import jax.numpy as jnp
from jax import lax
from jax.experimental import pallas as pl
from jax.experimental.pallas import tpu as pltpu

E = 8
D = 1024
H = 2048
O = 1024
T = 2048
G = 64

BH = 256
NG = H // BH
BO = O // NG


def _dot(a, b):
    return lax.dot_general(a, b, (((1,), (0,)), ((), ())),
                           preferred_element_type=jnp.float32)


def _moe_body(x_ref, xT_ref, W1_ref, b1_ref, W2_ref, b2_ref, W3_ref, b3_ref,
              Wg1_ref, bg1_ref, Wg2_ref, bg2_ref, out_ref,
              gates_ref, ge_ref, h1_ref, h2_ref):
    e = pl.program_id(0)
    p = pl.program_id(1)
    g = pl.program_id(2)

    @pl.when(jnp.logical_and(e == 0, jnp.logical_and(p == 0, g == 0)))
    def _compute_gates():
        gh = jnp.maximum(_dot(Wg1_ref[...], x_ref[...]) + bg1_ref[...], 0.0)
        logits = _dot(Wg2_ref[...], gh) + bg2_ref[...]
        m = jnp.max(logits, axis=0, keepdims=True)
        pr = jnp.exp(logits - m)
        gates_ref[...] = pr / jnp.sum(pr, axis=0, keepdims=True)

    @pl.when(jnp.logical_and(p == 0, g == 0))
    def _extract_gate():
        row = lax.broadcasted_iota(jnp.int32, (E, 1), 0)
        ge_ref[...] = jnp.sum(
            gates_ref[...] * (row == e).astype(jnp.float32),
            axis=0, keepdims=True)

    @pl.when(p == 0)
    def _layer1():
        h1c = jnp.maximum(
            _dot(W1_ref[0].astype(jnp.bfloat16), xT_ref[...]) + b1_ref[0],
            0.0)
        h1_ref[pl.ds(g * BH, BH), :] = h1c.astype(jnp.bfloat16)

    @pl.when(p == 1)
    def _layer2():
        h2c = jnp.maximum(
            _dot(W2_ref[0].astype(jnp.bfloat16), h1_ref[...]) + b2_ref[0],
            0.0) * ge_ref[...]
        h2_ref[pl.ds(g * BH, BH), :] = h2c.astype(jnp.bfloat16)

    @pl.when(p == 2)
    def _layer3():
        oc = _dot(W3_ref[0].astype(jnp.bfloat16), h2_ref[...])
        contrib = oc + ge_ref[...] * b3_ref[0]

        @pl.when(e == 0)
        def _init():
            out_ref[pl.ds(g * BO, BO), :] = contrib

        @pl.when(e != 0)
        def _accum():
            out_ref[pl.ds(g * BO, BO), :] += contrib


def _moe_pallas(x, W1, b1, W2, b2, W3, b3, Wg1, bg1, Wg2, bg2,
                interpret=False):
    t_tok = x.shape[0]
    xT = x.T.astype(jnp.bfloat16)
    xf = x.T
    b1r = b1[:, :, None]
    b2r = b2[:, :, None]
    b3r = b3[:, :, None]
    bg1r = bg1[:, None]
    bg2r = bg2[:, None]
    grid = (E, 3, NG)

    def w1_idx(e, p, g):
        return (e, jnp.where(p == 0, g, NG - 1), 0)

    def w2_idx(e, p, g):
        return (e, jnp.where(p == 1, g, jnp.where(p == 0, 0, NG - 1)), 0)

    def w3_idx(e, p, g):
        return (e, jnp.where(p == 2, g, 0), 0)

    outT = pl.pallas_call(
        _moe_body,
        grid=grid,
        in_specs=[
            pl.BlockSpec((D, t_tok), lambda e, p, g: (0, 0)),
            pl.BlockSpec((D, t_tok), lambda e, p, g: (0, 0)),
            pl.BlockSpec((1, BH, D), w1_idx),
            pl.BlockSpec((1, BH, 1), lambda e, p, g:
                         (e, jnp.where(p == 0, g, NG - 1), 0)),
            pl.BlockSpec((1, BH, H), w2_idx),
            pl.BlockSpec((1, BH, 1), lambda e, p, g:
                         (e, jnp.where(p == 1, g,
                                       jnp.where(p == 0, 0, NG - 1)), 0)),
            pl.BlockSpec((1, BO, H), w3_idx),
            pl.BlockSpec((1, BO, 1), lambda e, p, g:
                         (e, jnp.where(p == 2, g, 0), 0)),
            pl.BlockSpec((G, D), lambda e, p, g: (0, 0)),
            pl.BlockSpec((G, 1), lambda e, p, g: (0, 0)),
            pl.BlockSpec((E, G), lambda e, p, g: (0, 0)),
            pl.BlockSpec((E, 1), lambda e, p, g: (0, 0)),
        ],
        out_specs=pl.BlockSpec((O, t_tok), lambda e, p, g: (0, 0)),
        out_shape=jax.ShapeDtypeStruct((O, t_tok), jnp.float32),
        scratch_shapes=[
            pltpu.VMEM((E, t_tok), jnp.float32),
            pltpu.VMEM((1, t_tok), jnp.float32),
            pltpu.VMEM((H, t_tok), jnp.bfloat16),
            pltpu.VMEM((H, t_tok), jnp.bfloat16),
        ],
        interpret=interpret,
    )(xf, xT, W1, b1r, W2, b2r, W3, b3r, Wg1, bg1r, Wg2, bg2r)
    return outT.T


@jax.jit
def kernel(x, W1, b1, W2, b2, W3, b3, Wg1, bg1, Wg2, bg2):
    return _moe_pallas(x, W1, b1, W2, b2, W3, b3, Wg1, bg1, Wg2, bg2)


# speedup vs baseline: 1.8283x; 1.1198x over previous
import jax
import jax.numpy as jnp
from jax import lax
from jax.experimental import pallas as pl
from jax.experimental.pallas import tpu as pltpu

E = 8
D = 1024
H = 2048
O = 1024
T = 2048
G = 64

BH = 512
NG = H // BH
BO = O // NG


def _dot(a, b):
    return lax.dot_general(a, b, (((1,), (0,)), ((), ())),
                           preferred_element_type=jnp.float32)


def _moe_body(xT_ref, W1_ref, b1_ref, W2_ref, b2_ref, W3_ref, b3_ref,
              Wg1_ref, bg1_ref, Wg2_ref, bg2_ref, out_ref,
              gates_ref, ge_ref, h1_ref, h2_ref):
    e = pl.program_id(0)
    p = pl.program_id(1)
    g = pl.program_id(2)

    @pl.when(jnp.logical_and(e == 0, jnp.logical_and(p == 0, g == 0)))
    def _compute_gates():
        gh = jnp.maximum(_dot(Wg1_ref[...].astype(jnp.bfloat16),
                              xT_ref[...]) + bg1_ref[...], 0.0)
        logits = _dot(Wg2_ref[...], gh) + bg2_ref[...]
        m = jnp.max(logits, axis=0, keepdims=True)
        pr = jnp.exp(logits - m)
        gates_ref[...] = pr / jnp.sum(pr, axis=0, keepdims=True)

    @pl.when(jnp.logical_and(p == 0, g == 0))
    def _extract_gate():
        row = lax.broadcasted_iota(jnp.int32, (E, 1), 0)
        ge_ref[...] = jnp.sum(
            gates_ref[...] * (row == e).astype(jnp.float32),
            axis=0, keepdims=True)

    @pl.when(p == 0)
    def _layer1():
        h1c = jnp.maximum(
            _dot(W1_ref[0].astype(jnp.bfloat16), xT_ref[...]) + b1_ref[0],
            0.0)
        h1_ref[pl.ds(g * BH, BH), :] = h1c.astype(jnp.bfloat16)

    @pl.when(p == 1)
    def _layer2():
        h2c = jnp.maximum(
            _dot(W2_ref[0].astype(jnp.bfloat16), h1_ref[...]) + b2_ref[0],
            0.0) * ge_ref[...]
        h2_ref[pl.ds(g * BH, BH), :] = h2c.astype(jnp.bfloat16)

    @pl.when(p == 2)
    def _layer3():
        oc = _dot(W3_ref[0].astype(jnp.bfloat16), h2_ref[...])
        contrib = oc + ge_ref[...] * b3_ref[0]

        @pl.when(e == 0)
        def _init():
            out_ref[pl.ds(g * BO, BO), :] = contrib

        @pl.when(e != 0)
        def _accum():
            out_ref[pl.ds(g * BO, BO), :] += contrib


def _moe_pallas(x, W1, b1, W2, b2, W3, b3, Wg1, bg1, Wg2, bg2,
                interpret=False):
    t_tok = x.shape[0]
    xT = x.T.astype(jnp.bfloat16)
    b1r = b1[:, :, None]
    b2r = b2[:, :, None]
    b3r = b3[:, :, None]
    bg1r = bg1[:, None]
    bg2r = bg2[:, None]
    grid = (E, 3, NG)

    def w1_idx(e, p, g):
        return (e, jnp.where(p == 0, g, NG - 1), 0)

    def w2_idx(e, p, g):
        return (e, jnp.where(p == 1, g, jnp.where(p == 0, 0, NG - 1)), 0)

    def w3_idx(e, p, g):
        return (e, jnp.where(p == 2, g, 0), 0)

    outT = pl.pallas_call(
        _moe_body,
        grid=grid,
        in_specs=[
            pl.BlockSpec((D, t_tok), lambda e, p, g: (0, 0)),
            pl.BlockSpec((1, BH, D), w1_idx),
            pl.BlockSpec((1, BH, 1), lambda e, p, g:
                         (e, jnp.where(p == 0, g, NG - 1), 0)),
            pl.BlockSpec((1, BH, H), w2_idx),
            pl.BlockSpec((1, BH, 1), lambda e, p, g:
                         (e, jnp.where(p == 1, g,
                                       jnp.where(p == 0, 0, NG - 1)), 0)),
            pl.BlockSpec((1, BO, H), w3_idx),
            pl.BlockSpec((1, BO, 1), lambda e, p, g:
                         (e, jnp.where(p == 2, g, 0), 0)),
            pl.BlockSpec((G, D), lambda e, p, g: (0, 0)),
            pl.BlockSpec((G, 1), lambda e, p, g: (0, 0)),
            pl.BlockSpec((E, G), lambda e, p, g: (0, 0)),
            pl.BlockSpec((E, 1), lambda e, p, g: (0, 0)),
        ],
        out_specs=pl.BlockSpec((O, t_tok), lambda e, p, g: (0, 0)),
        out_shape=jax.ShapeDtypeStruct((O, t_tok), jnp.float32),
        scratch_shapes=[
            pltpu.VMEM((E, t_tok), jnp.float32),
            pltpu.VMEM((1, t_tok), jnp.float32),
            pltpu.VMEM((H, t_tok), jnp.bfloat16),
            pltpu.VMEM((H, t_tok), jnp.bfloat16),
        ],
        interpret=interpret,
    )(xT, W1, b1r, W2, b2r, W3, b3r, Wg1, bg1r, Wg2, bg2r)
    return outT.T


@jax.jit
def kernel(x, W1, b1, W2, b2, W3, b3, Wg1, bg1, Wg2, bg2):
    return _moe_pallas(x, W1, b1, W2, b2, W3, b3, Wg1, bg1, Wg2, bg2)
